# baseline (speedup 1.0000x reference)
import functools

import jax
import jax.numpy as jnp
import numpy as np
from jax import lax
from jax.experimental import pallas as pl
from jax.experimental.pallas import tpu as pltpu
from jax.experimental.pallas import tpu_sc as plsc

N = 10000
E = 320000
D = 128
N_CLS = 2

NPAD = 10240
NC = 2
NS = 16
NW = NC * NS
CH = 128
CPT = 80
HPT = CPT // 2
EPT = CH * CPT
EPAD = EPT * NW
SROWS = NPAD // NS
SBLK = SROWS // CH

f32 = jnp.float32
i32 = jnp.int32


def _mesh():
    return plsc.VectorSubcoreMesh(core_axis_name="c", subcore_axis_name="s")


def _zero_rows(ref, nrows, width):
    z = jnp.zeros((16,), f32)

    def body(i, _):
        for k in range(width // 16):
            ref[i, pl.ds(k * 16, 16)] = z
        return 0

    lax.fori_loop(0, nrows, body, 0)


def _deg_kernel(dstr, degp, ones_v, idx_v, tmp_v, deg_sh):
    c = lax.axis_index("c")
    s = lax.axis_index("s")
    w = c * NS + s

    one = jnp.ones((16,), f32)

    def fill(i, _):
        ones_v[i, pl.ds(0, 16)] = one
        tmp_v[i, pl.ds(0, 16)] = jnp.zeros((16,), f32)
        return 0

    lax.fori_loop(0, CH, fill, 0)

    for k in range(SBLK):
        pltpu.sync_copy(tmp_v, deg_sh.at[pl.ds(s * SROWS + k * CH, CH)])
    plsc.subcore_barrier()

    pltpu.sync_copy(dstr.at[w], idx_v)

    def chunk(j, _):
        pltpu.sync_copy(ones_v, deg_sh.at[idx_v.at[j]], add=True)
        return 0

    lax.fori_loop(0, CPT, chunk, 0)
    plsc.subcore_barrier()

    for k in range(SBLK):
        r0 = s * SROWS + k * CH
        pltpu.sync_copy(deg_sh.at[pl.ds(r0, CH)], tmp_v)
        pltpu.sync_copy(tmp_v, degp.at[c, pl.ds(r0, CH)])


def _deg_call(dstr):
    return pl.kernel(
        _deg_kernel,
        out_type=jax.ShapeDtypeStruct((NC, NPAD, 16), f32),
        mesh=_mesh(),
        scratch_types=[
            pltpu.VMEM((CH, 16), f32),
            pltpu.VMEM((CPT, CH), i32),
            pltpu.VMEM((CH, 16), f32),
            pltpu.VMEM_SHARED((NPAD, 16), f32),
        ],
        name="sc_degree",
    )(dstr)


def _agg_kernel(y, srcr, dstr, aggp, sidx_v, didx_v, rows0, rows1, sem0, sem1,
                agg_sh):
    c = lax.axis_index("c")
    s = lax.axis_index("s")
    w = c * NS + s

    _zero_rows(rows0, CH, D)
    for k in range(SBLK):
        pltpu.sync_copy(rows0, agg_sh.at[pl.ds(s * SROWS + k * CH, CH)])
    plsc.subcore_barrier()

    rows = (rows0, rows1)
    sems = (sem0, sem1)

    for phase in range(2):
        base = phase * HPT
        pltpu.sync_copy(srcr.at[w, pl.ds(base, HPT)], sidx_v)
        pltpu.sync_copy(dstr.at[w, pl.ds(base, HPT)], didx_v)

        pltpu.async_copy(y.at[sidx_v.at[0]], rows0, sem0)
        pltpu.async_copy(y.at[sidx_v.at[1]], rows1, sem1)

        def pair(i, _):
            for b in range(2):
                j = 2 * i + b
                r, sem = rows[b], sems[b]
                pltpu.make_async_copy(y.at[sidx_v.at[j]], r, sem).wait()
                pltpu.sync_copy(r, agg_sh.at[didx_v.at[j]], add=True)
                jn = jnp.minimum(j + 2, HPT - 1)
                pltpu.async_copy(y.at[sidx_v.at[jn]], r, sem)
            return 0

        lax.fori_loop(0, HPT // 2, pair, 0)
        pltpu.make_async_copy(y.at[sidx_v.at[HPT - 1]], rows0, sem0).wait()
        pltpu.make_async_copy(y.at[sidx_v.at[HPT - 1]], rows1, sem1).wait()

    plsc.subcore_barrier()

    for k in range(SBLK):
        r0 = s * SROWS + k * CH
        pltpu.sync_copy(agg_sh.at[pl.ds(r0, CH)], rows0)
        pltpu.sync_copy(rows0, aggp.at[c, pl.ds(r0, CH)])


def _agg_call(y, srcr, dstr):
    return pl.kernel(
        _agg_kernel,
        out_type=jax.ShapeDtypeStruct((NC, NPAD, D), f32),
        mesh=_mesh(),
        scratch_types=[
            pltpu.VMEM((HPT, CH), i32),
            pltpu.VMEM((HPT, CH), i32),
            pltpu.VMEM((CH, D), f32),
            pltpu.VMEM((CH, D), f32),
            pltpu.SemaphoreType.DMA,
            pltpu.SemaphoreType.DMA,
            pltpu.VMEM_SHARED((NPAD, D), f32),
        ],
        name="sc_conv_agg",
    )(y, srcr, dstr)


def _edge_kernel(a1, a2, srcr, dstr, w0, w1, pout,
                 sidx_v, didx_v, g1a, g2a, g1b, g2b, sema, semb,
                 w0_v, w1_v, p_v):
    c = lax.axis_index("c")
    s = lax.axis_index("s")
    w = c * NS + s

    pltpu.sync_copy(w0, w0_v)
    pltpu.sync_copy(w1, w1_v)
    pltpu.sync_copy(srcr.at[w], sidx_v)
    pltpu.sync_copy(dstr.at[w], didx_v)

    g1s = (g1a, g1b)
    g2s = (g2a, g2b)
    sems = (sema, semb)

    w0s = tuple(w0_v[pl.ds(k * 16, 16)] for k in range(D // 16))
    w1s = tuple(w1_v[pl.ds(k * 16, 16)] for k in range(D // 16))

    pltpu.async_copy(a1.at[sidx_v.at[0]], g1a, sema)
    pltpu.async_copy(a2.at[didx_v.at[0]], g2a, sema)
    pltpu.async_copy(a1.at[sidx_v.at[1]], g1b, semb)
    pltpu.async_copy(a2.at[didx_v.at[1]], g2b, semb)

    def pair(i, _):
        for b in range(2):
            j = 2 * i + b
            g1_v, g2_v, sem = g1s[b], g2s[b], sems[b]
            pltpu.make_async_copy(a1.at[sidx_v.at[j]], g1_v, sem).wait()
            pltpu.make_async_copy(a2.at[didx_v.at[j]], g2_v, sem).wait()

            def edge(e, _):
                for k in range(D // 16):
                    sl = pl.ds(k * 16, 16)
                    v = jnp.maximum(g1_v[e, sl] + g2_v[e, sl], 0.0)
                    if k == 0:
                        a0e = v * w0s[k]
                        a1e = v * w1s[k]
                    elif k == 1:
                        a0o = v * w0s[k]
                        a1o = v * w1s[k]
                    elif k % 2 == 0:
                        a0e = a0e + v * w0s[k]
                        a1e = a1e + v * w1s[k]
                    else:
                        a0o = a0o + v * w0s[k]
                        a1o = a1o + v * w1s[k]
                p_v[pl.ds(e * 32, 16)] = a0e + a0o
                p_v[pl.ds(e * 32 + 16, 16)] = a1e + a1o
                return 0

            lax.fori_loop(0, CH, edge, 0)

            pltpu.sync_copy(p_v, pout.at[w * CPT + j])
            jn = jnp.minimum(j + 2, CPT - 1)
            pltpu.async_copy(a1.at[sidx_v.at[jn]], g1_v, sem)
            pltpu.async_copy(a2.at[didx_v.at[jn]], g2_v, sem)
        return 0

    lax.fori_loop(0, CPT // 2, pair, 0)
    pltpu.make_async_copy(a1.at[sidx_v.at[CPT - 1]], g1a, sema).wait()
    pltpu.make_async_copy(a2.at[didx_v.at[CPT - 1]], g2a, sema).wait()
    pltpu.make_async_copy(a1.at[sidx_v.at[CPT - 1]], g1b, semb).wait()
    pltpu.make_async_copy(a2.at[didx_v.at[CPT - 1]], g2b, semb).wait()


def _edge_call(a1, a2, srcr, dstr, w0, w1):
    return pl.kernel(
        _edge_kernel,
        out_type=jax.ShapeDtypeStruct((NW * CPT, 32 * CH), f32),
        mesh=_mesh(),
        scratch_types=[
            pltpu.VMEM((CPT, CH), i32),
            pltpu.VMEM((CPT, CH), i32),
            pltpu.VMEM((CH, D), f32),
            pltpu.VMEM((CH, D), f32),
            pltpu.VMEM((CH, D), f32),
            pltpu.VMEM((CH, D), f32),
            pltpu.SemaphoreType.DMA,
            pltpu.SemaphoreType.DMA,
            pltpu.VMEM((D,), f32),
            pltpu.VMEM((D,), f32),
            pltpu.VMEM((32 * CH,), f32),
        ],
        name="sc_edge_head",
    )(a1, a2, srcr, dstr, w0, w1)


_RB = 1024


def _k1_body(x_ref, w_ref, d0_ref, d1_ref, y_ref, dis_ref):
    deg = d0_ref[:, 0:1] + d1_ref[:, 0:1] + 1.0
    dis = lax.rsqrt(deg)
    dis_ref[...] = dis
    xw = jnp.dot(x_ref[...], w_ref[...], preferred_element_type=f32)
    y_ref[...] = xw * dis


def _k1_call(x_pad, W1, d0, d1):
    grid = NPAD // _RB
    return pl.pallas_call(
        _k1_body,
        grid=(grid,),
        in_specs=[
            pl.BlockSpec((_RB, D), lambda i: (i, 0)),
            pl.BlockSpec((D, D), lambda i: (0, 0)),
            pl.BlockSpec((_RB, 16), lambda i: (i, 0)),
            pl.BlockSpec((_RB, 16), lambda i: (i, 0)),
        ],
        out_specs=[
            pl.BlockSpec((_RB, D), lambda i: (i, 0)),
            pl.BlockSpec((_RB, 1), lambda i: (i, 0)),
        ],
        out_shape=[
            jax.ShapeDtypeStruct((NPAD, D), f32),
            jax.ShapeDtypeStruct((NPAD, 1), f32),
        ],
        name="tc_y1_dis",
    )(x_pad, W1, d0, d1)


def _k2_body(aa_ref, ab_ref, y_ref, dis_ref, w_ref, b_ref, y2_ref):
    dis = dis_ref[...]
    t = (aa_ref[...] + ab_ref[...] + y_ref[...]) * dis + b_ref[...]
    h = jnp.maximum(t, 0.0)
    y2_ref[...] = jnp.dot(h, w_ref[...], preferred_element_type=f32) * dis


def _k2_call(agga, aggb, y1, dis, W2, b1row):
    grid = NPAD // _RB
    return pl.pallas_call(
        _k2_body,
        grid=(grid,),
        in_specs=[
            pl.BlockSpec((_RB, D), lambda i: (i, 0)),
            pl.BlockSpec((_RB, D), lambda i: (i, 0)),
            pl.BlockSpec((_RB, D), lambda i: (i, 0)),
            pl.BlockSpec((_RB, 1), lambda i: (i, 0)),
            pl.BlockSpec((D, D), lambda i: (0, 0)),
            pl.BlockSpec((1, D), lambda i: (0, 0)),
        ],
        out_specs=pl.BlockSpec((_RB, D), lambda i: (i, 0)),
        out_shape=jax.ShapeDtypeStruct((NPAD, D), f32),
        name="tc_y2",
    )(agga, aggb, y1, dis, W2, b1row)


def _k3_body(aa_ref, ab_ref, y_ref, dis_ref, b_ref, wt_ref, wb_ref, blr_ref,
             a1_ref, a2_ref):
    t = (aa_ref[...] + ab_ref[...] + y_ref[...]) * dis_ref[...] + b_ref[...]
    h = jnp.maximum(t, 0.0)
    a1_ref[...] = jnp.dot(h, wt_ref[...], preferred_element_type=f32) + blr_ref[...]
    a2_ref[...] = jnp.dot(h, wb_ref[...], preferred_element_type=f32)


def _k3_call(agga, aggb, y2, dis, b2row, Wl1t, Wl1b, bl1row):
    grid = NPAD // _RB
    return pl.pallas_call(
        _k3_body,
        grid=(grid,),
        in_specs=[
            pl.BlockSpec((_RB, D), lambda i: (i, 0)),
            pl.BlockSpec((_RB, D), lambda i: (i, 0)),
            pl.BlockSpec((_RB, D), lambda i: (i, 0)),
            pl.BlockSpec((_RB, 1), lambda i: (i, 0)),
            pl.BlockSpec((1, D), lambda i: (0, 0)),
            pl.BlockSpec((D, D), lambda i: (0, 0)),
            pl.BlockSpec((D, D), lambda i: (0, 0)),
            pl.BlockSpec((1, D), lambda i: (0, 0)),
        ],
        out_specs=[
            pl.BlockSpec((_RB, D), lambda i: (i, 0)),
            pl.BlockSpec((_RB, D), lambda i: (i, 0)),
        ],
        out_shape=[
            jax.ShapeDtypeStruct((NPAD, D), f32),
            jax.ShapeDtypeStruct((NPAD, D), f32),
        ],
        name="tc_a1_a2",
    )(agga, aggb, y2, dis, b2row, Wl1t, Wl1b, bl1row)


_EB = 4096


_CB = 64

_M_RED = np.zeros((32 * CH, N_CLS * CH), np.float32)
for _e in range(CH):
    for _c in range(N_CLS):
        _M_RED[_e * 32 + _c * 16:_e * 32 + _c * 16 + 16, _e * N_CLS + _c] = 1.0


def _k4_body(p_ref, m_ref, bl2_ref, o_ref):
    s = jnp.dot(p_ref[...], m_ref[...], preferred_element_type=f32)
    o_ref[...] = jnp.maximum(s + bl2_ref[...], 0.0)


def _k4_call(p, m, bl2row):
    grid = (NW * CPT) // _CB
    return pl.pallas_call(
        _k4_body,
        grid=(grid,),
        in_specs=[
            pl.BlockSpec((_CB, 32 * CH), lambda i: (i, 0)),
            pl.BlockSpec((32 * CH, N_CLS * CH), lambda i: (0, 0)),
            pl.BlockSpec((1, N_CLS * CH), lambda i: (0, 0)),
        ],
        out_specs=pl.BlockSpec((_CB, N_CLS * CH), lambda i: (i, 0)),
        out_shape=jax.ShapeDtypeStruct((NW * CPT, N_CLS * CH), f32),
        name="tc_logits",
    )(p, m, bl2row)


def kernel(x, edge_index, W1, b1, W2, b2, Wl1, bl1, Wl2, bl2):
    x_pad = jnp.zeros((NPAD, D), f32).at[:N].set(x)

    src = edge_index[0]
    dst = edge_index[1]
    pad = N + (jnp.arange(EPAD - E, dtype=i32) % (NPAD - N))
    srcr = jnp.concatenate([src, pad]).reshape(NW, CPT, CH)
    dstr = jnp.concatenate([dst, pad]).reshape(NW, CPT, CH)

    degp = _deg_call(dstr)
    d0 = degp[0]
    d1 = degp[1]

    y1, dis = _k1_call(x_pad, W1, d0, d1)

    agg1 = _agg_call(y1, srcr, dstr)
    y2 = _k2_call(agg1[0], agg1[1], y1, dis, W2, b1.reshape(1, D))

    agg2 = _agg_call(y2, srcr, dstr)
    a1, a2 = _k3_call(agg2[0], agg2[1], y2, dis, b2.reshape(1, D),
                      Wl1[:D], Wl1[D:], bl1.reshape(1, D))

    p = _edge_call(a1, a2, srcr, dstr,
                   Wl2[:, 0] + 0.0,
                   Wl2[:, 1] + 0.0)

    bl2row = jnp.tile(bl2, (CH,)).reshape(1, N_CLS * CH)
    out = _k4_call(p, jnp.asarray(_M_RED), bl2row)
    return out.reshape(EPAD, N_CLS)[:E]


# speedup vs baseline: 1.1649x; 1.1649x over previous
import functools

import jax
import jax.numpy as jnp
import numpy as np
from jax import lax
from jax.experimental import pallas as pl
from jax.experimental.pallas import tpu as pltpu
from jax.experimental.pallas import tpu_sc as plsc

N = 10000
E = 320000
D = 128
N_CLS = 2

NPAD = 10240
NC = 2
NS = 16
NW = NC * NS
CH = 128
CPT = 80
HPT = CPT // 2
EPT = CH * CPT
EPAD = EPT * NW
SROWS = NPAD // NS
SBLK = SROWS // CH

f32 = jnp.float32
i32 = jnp.int32


def _mesh():
    return plsc.VectorSubcoreMesh(core_axis_name="c", subcore_axis_name="s")


def _zero_rows(ref, nrows, width):
    z = jnp.zeros((16,), f32)

    def body(i, _):
        for k in range(width // 16):
            ref[i, pl.ds(k * 16, 16)] = z
        return 0

    lax.fori_loop(0, nrows, body, 0)


def _deg_kernel(dstr, degp, ones_v, idx_v, tmp_v, deg_sh):
    c = lax.axis_index("c")
    s = lax.axis_index("s")
    w = c * NS + s

    one = jnp.ones((16,), f32)

    def fill(i, _):
        ones_v[i, pl.ds(0, 16)] = one
        tmp_v[i, pl.ds(0, 16)] = jnp.zeros((16,), f32)
        return 0

    lax.fori_loop(0, CH, fill, 0)

    for k in range(SBLK):
        pltpu.sync_copy(tmp_v, deg_sh.at[pl.ds(s * SROWS + k * CH, CH)])
    plsc.subcore_barrier()

    pltpu.sync_copy(dstr.at[w], idx_v)

    def chunk(j, _):
        pltpu.sync_copy(ones_v, deg_sh.at[idx_v.at[j]], add=True)
        return 0

    lax.fori_loop(0, CPT, chunk, 0)
    plsc.subcore_barrier()

    for k in range(SBLK):
        r0 = s * SROWS + k * CH
        pltpu.sync_copy(deg_sh.at[pl.ds(r0, CH)], tmp_v)
        pltpu.sync_copy(tmp_v, degp.at[c, pl.ds(r0, CH)])


def _deg_call(dstr):
    return pl.kernel(
        _deg_kernel,
        out_type=jax.ShapeDtypeStruct((NC, NPAD, 16), f32),
        mesh=_mesh(),
        scratch_types=[
            pltpu.VMEM((CH, 16), f32),
            pltpu.VMEM((CPT, CH), i32),
            pltpu.VMEM((CH, 16), f32),
            pltpu.VMEM_SHARED((NPAD, 16), f32),
        ],
        name="sc_degree",
    )(dstr)


def _agg_kernel(y, srcr, dstr, aggp, sidx_v, didx_v, rows0, rows1, sem0, sem1,
                agg_sh):
    c = lax.axis_index("c")
    s = lax.axis_index("s")
    w = c * NS + s

    _zero_rows(rows0, CH, D)
    for k in range(SBLK):
        pltpu.sync_copy(rows0, agg_sh.at[pl.ds(s * SROWS + k * CH, CH)])
    plsc.subcore_barrier()

    rows = (rows0, rows1)
    sems = (sem0, sem1)

    for phase in range(2):
        base = phase * HPT
        pltpu.sync_copy(srcr.at[w, pl.ds(base, HPT)], sidx_v)
        pltpu.sync_copy(dstr.at[w, pl.ds(base, HPT)], didx_v)

        pltpu.async_copy(y.at[sidx_v.at[0]], rows0, sem0)
        pltpu.async_copy(y.at[sidx_v.at[1]], rows1, sem1)

        def pair(i, _):
            for b in range(2):
                j = 2 * i + b
                r, sem = rows[b], sems[b]
                pltpu.make_async_copy(y.at[sidx_v.at[j]], r, sem).wait()
                pltpu.sync_copy(r, agg_sh.at[didx_v.at[j]], add=True)
                jn = jnp.minimum(j + 2, HPT - 1)
                pltpu.async_copy(y.at[sidx_v.at[jn]], r, sem)
            return 0

        lax.fori_loop(0, HPT // 2, pair, 0)
        pltpu.make_async_copy(y.at[sidx_v.at[HPT - 1]], rows0, sem0).wait()
        pltpu.make_async_copy(y.at[sidx_v.at[HPT - 1]], rows1, sem1).wait()

    plsc.subcore_barrier()

    for k in range(SBLK):
        r0 = s * SROWS + k * CH
        pltpu.sync_copy(agg_sh.at[pl.ds(r0, CH)], rows0)
        pltpu.sync_copy(rows0, aggp.at[c, pl.ds(r0, CH)])


def _agg_call(y, srcr, dstr):
    return pl.kernel(
        _agg_kernel,
        out_type=jax.ShapeDtypeStruct((NC, NPAD, D), f32),
        mesh=_mesh(),
        scratch_types=[
            pltpu.VMEM((HPT, CH), i32),
            pltpu.VMEM((HPT, CH), i32),
            pltpu.VMEM((CH, D), f32),
            pltpu.VMEM((CH, D), f32),
            pltpu.SemaphoreType.DMA,
            pltpu.SemaphoreType.DMA,
            pltpu.VMEM_SHARED((NPAD, D), f32),
        ],
        name="sc_conv_agg",
    )(y, srcr, dstr)


def _edge_kernel(a1, a2, srcr, dstr, w0, w1, pout,
                 sidx_v, didx_v, g1a, g2a, g1b, g2b, sema, semb,
                 w0_v, w1_v, p_v):
    c = lax.axis_index("c")
    s = lax.axis_index("s")
    w = c * NS + s

    pltpu.sync_copy(w0, w0_v)
    pltpu.sync_copy(w1, w1_v)
    pltpu.sync_copy(srcr.at[w], sidx_v)
    pltpu.sync_copy(dstr.at[w], didx_v)

    g1s = (g1a, g1b)
    g2s = (g2a, g2b)
    sems = (sema, semb)

    w0s = tuple(w0_v[pl.ds(k * 16, 16)] for k in range(D // 16))
    w1s = tuple(w1_v[pl.ds(k * 16, 16)] for k in range(D // 16))

    pltpu.async_copy(a1.at[sidx_v.at[0]], g1a, sema)
    pltpu.async_copy(a2.at[didx_v.at[0]], g2a, sema)
    pltpu.async_copy(a1.at[sidx_v.at[1]], g1b, semb)
    pltpu.async_copy(a2.at[didx_v.at[1]], g2b, semb)

    def pair(i, _):
        for b in range(2):
            j = 2 * i + b
            g1_v, g2_v, sem = g1s[b], g2s[b], sems[b]
            pltpu.make_async_copy(a1.at[sidx_v.at[j]], g1_v, sem).wait()
            pltpu.make_async_copy(a2.at[didx_v.at[j]], g2_v, sem).wait()

            def edge(e, _):
                for k in range(D // 16):
                    sl = pl.ds(k * 16, 16)
                    v = jnp.maximum(g1_v[e, sl] + g2_v[e, sl], 0.0)
                    if k == 0:
                        a0e = v * w0s[k]
                        a1e = v * w1s[k]
                    elif k == 1:
                        a0o = v * w0s[k]
                        a1o = v * w1s[k]
                    elif k % 2 == 0:
                        a0e = a0e + v * w0s[k]
                        a1e = a1e + v * w1s[k]
                    else:
                        a0o = a0o + v * w0s[k]
                        a1o = a1o + v * w1s[k]
                q = e >> 4
                rm = (e & 15) * 32
                p_v[q, pl.ds(rm, 16)] = a0e + a0o
                p_v[q, pl.ds(rm + 16, 16)] = a1e + a1o
                return 0

            lax.fori_loop(0, CH, edge, 0)

            pltpu.sync_copy(p_v, pout.at[pl.ds((w * CPT + j) * 8, 8)])
            jn = jnp.minimum(j + 2, CPT - 1)
            pltpu.async_copy(a1.at[sidx_v.at[jn]], g1_v, sem)
            pltpu.async_copy(a2.at[didx_v.at[jn]], g2_v, sem)
        return 0

    lax.fori_loop(0, CPT // 2, pair, 0)
    pltpu.make_async_copy(a1.at[sidx_v.at[CPT - 1]], g1a, sema).wait()
    pltpu.make_async_copy(a2.at[didx_v.at[CPT - 1]], g2a, sema).wait()
    pltpu.make_async_copy(a1.at[sidx_v.at[CPT - 1]], g1b, semb).wait()
    pltpu.make_async_copy(a2.at[didx_v.at[CPT - 1]], g2b, semb).wait()


def _edge_call(a1, a2, srcr, dstr, w0, w1):
    return pl.kernel(
        _edge_kernel,
        out_type=jax.ShapeDtypeStruct((NW * CPT * 8, 512), f32),
        mesh=_mesh(),
        scratch_types=[
            pltpu.VMEM((CPT, CH), i32),
            pltpu.VMEM((CPT, CH), i32),
            pltpu.VMEM((CH, D), f32),
            pltpu.VMEM((CH, D), f32),
            pltpu.VMEM((CH, D), f32),
            pltpu.VMEM((CH, D), f32),
            pltpu.SemaphoreType.DMA,
            pltpu.SemaphoreType.DMA,
            pltpu.VMEM((D,), f32),
            pltpu.VMEM((D,), f32),
            pltpu.VMEM((8, 512), f32),
        ],
        name="sc_edge_head",
    )(a1, a2, srcr, dstr, w0, w1)


_RB = 1024


def _k1_body(x_ref, w_ref, d0_ref, d1_ref, y_ref, dis_ref):
    deg = d0_ref[:, 0:1] + d1_ref[:, 0:1] + 1.0
    dis = lax.rsqrt(deg)
    dis_ref[...] = dis
    xw = jnp.dot(x_ref[...], w_ref[...], preferred_element_type=f32)
    y_ref[...] = xw * dis


def _k1_call(x_pad, W1, d0, d1):
    grid = NPAD // _RB
    return pl.pallas_call(
        _k1_body,
        grid=(grid,),
        in_specs=[
            pl.BlockSpec((_RB, D), lambda i: (i, 0)),
            pl.BlockSpec((D, D), lambda i: (0, 0)),
            pl.BlockSpec((_RB, 16), lambda i: (i, 0)),
            pl.BlockSpec((_RB, 16), lambda i: (i, 0)),
        ],
        out_specs=[
            pl.BlockSpec((_RB, D), lambda i: (i, 0)),
            pl.BlockSpec((_RB, 1), lambda i: (i, 0)),
        ],
        out_shape=[
            jax.ShapeDtypeStruct((NPAD, D), f32),
            jax.ShapeDtypeStruct((NPAD, 1), f32),
        ],
        name="tc_y1_dis",
    )(x_pad, W1, d0, d1)


def _k2_body(aa_ref, ab_ref, y_ref, dis_ref, w_ref, b_ref, y2_ref):
    dis = dis_ref[...]
    t = (aa_ref[...] + ab_ref[...] + y_ref[...]) * dis + b_ref[...]
    h = jnp.maximum(t, 0.0)
    y2_ref[...] = jnp.dot(h, w_ref[...], preferred_element_type=f32) * dis


def _k2_call(agga, aggb, y1, dis, W2, b1row):
    grid = NPAD // _RB
    return pl.pallas_call(
        _k2_body,
        grid=(grid,),
        in_specs=[
            pl.BlockSpec((_RB, D), lambda i: (i, 0)),
            pl.BlockSpec((_RB, D), lambda i: (i, 0)),
            pl.BlockSpec((_RB, D), lambda i: (i, 0)),
            pl.BlockSpec((_RB, 1), lambda i: (i, 0)),
            pl.BlockSpec((D, D), lambda i: (0, 0)),
            pl.BlockSpec((1, D), lambda i: (0, 0)),
        ],
        out_specs=pl.BlockSpec((_RB, D), lambda i: (i, 0)),
        out_shape=jax.ShapeDtypeStruct((NPAD, D), f32),
        name="tc_y2",
    )(agga, aggb, y1, dis, W2, b1row)


def _k3_body(aa_ref, ab_ref, y_ref, dis_ref, b_ref, wt_ref, wb_ref, blr_ref,
             a1_ref, a2_ref):
    t = (aa_ref[...] + ab_ref[...] + y_ref[...]) * dis_ref[...] + b_ref[...]
    h = jnp.maximum(t, 0.0)
    a1_ref[...] = jnp.dot(h, wt_ref[...], preferred_element_type=f32) + blr_ref[...]
    a2_ref[...] = jnp.dot(h, wb_ref[...], preferred_element_type=f32)


def _k3_call(agga, aggb, y2, dis, b2row, Wl1t, Wl1b, bl1row):
    grid = NPAD // _RB
    return pl.pallas_call(
        _k3_body,
        grid=(grid,),
        in_specs=[
            pl.BlockSpec((_RB, D), lambda i: (i, 0)),
            pl.BlockSpec((_RB, D), lambda i: (i, 0)),
            pl.BlockSpec((_RB, D), lambda i: (i, 0)),
            pl.BlockSpec((_RB, 1), lambda i: (i, 0)),
            pl.BlockSpec((1, D), lambda i: (0, 0)),
            pl.BlockSpec((D, D), lambda i: (0, 0)),
            pl.BlockSpec((D, D), lambda i: (0, 0)),
            pl.BlockSpec((1, D), lambda i: (0, 0)),
        ],
        out_specs=[
            pl.BlockSpec((_RB, D), lambda i: (i, 0)),
            pl.BlockSpec((_RB, D), lambda i: (i, 0)),
        ],
        out_shape=[
            jax.ShapeDtypeStruct((NPAD, D), f32),
            jax.ShapeDtypeStruct((NPAD, D), f32),
        ],
        name="tc_a1_a2",
    )(agga, aggb, y2, dis, b2row, Wl1t, Wl1b, bl1row)


_EB = 4096


_CB = 50

_M_RED = np.zeros((512, 32), np.float32)
for _e in range(16):
    for _c in range(N_CLS):
        _M_RED[_e * 32 + _c * 16:_e * 32 + _c * 16 + 16, _e * N_CLS + _c] = 1.0


_RB4 = 400


def _k4_body(p_ref, m_ref, bl2_ref, o_ref):
    s = jnp.dot(p_ref[...], m_ref[...], preferred_element_type=f32)
    o_ref[...] = jnp.maximum(s + bl2_ref[...], 0.0)


def _k4_call(p, m, bl2row):
    grid = (E // 16) // _RB4
    return pl.pallas_call(
        _k4_body,
        grid=(grid,),
        in_specs=[
            pl.BlockSpec((_RB4, 512), lambda i: (i, 0)),
            pl.BlockSpec((512, 32), lambda i: (0, 0)),
            pl.BlockSpec((1, 32), lambda i: (0, 0)),
        ],
        out_specs=pl.BlockSpec((_RB4, 32), lambda i: (i, 0)),
        out_shape=jax.ShapeDtypeStruct((E // 16, 32), f32),
        name="tc_logits",
    )(p, m, bl2row)


def kernel(x, edge_index, W1, b1, W2, b2, Wl1, bl1, Wl2, bl2):
    x_pad = jnp.zeros((NPAD, D), f32).at[:N].set(x)

    src = edge_index[0]
    dst = edge_index[1]
    pad = N + (jnp.arange(EPAD - E, dtype=i32) % (NPAD - N))
    srcr = jnp.concatenate([src, pad]).reshape(NW, CPT, CH)
    dstr = jnp.concatenate([dst, pad]).reshape(NW, CPT, CH)

    degp = _deg_call(dstr)
    d0 = degp[0]
    d1 = degp[1]

    y1, dis = _k1_call(x_pad, W1, d0, d1)

    agg1 = _agg_call(y1, srcr, dstr)
    y2 = _k2_call(agg1[0], agg1[1], y1, dis, W2, b1.reshape(1, D))

    agg2 = _agg_call(y2, srcr, dstr)
    a1, a2 = _k3_call(agg2[0], agg2[1], y2, dis, b2.reshape(1, D),
                      Wl1[:D], Wl1[D:], bl1.reshape(1, D))

    p = _edge_call(a1, a2, srcr, dstr,
                   Wl2[:, 0] + 0.0,
                   Wl2[:, 1] + 0.0)

    bl2row = jnp.tile(bl2, (16,)).reshape(1, 32)
    out = _k4_call(p, jnp.asarray(_M_RED), bl2row)
    return out.reshape(E, N_CLS)
